# baseline (speedup 1.0000x reference)
import functools

import jax
import jax.numpy as jnp
from jax import lax
from jax.experimental import pallas as pl
from jax.experimental.pallas import tpu as pltpu
from jax.experimental.pallas import tpu_sc as plsc

N = 10000
E = 320000
D = 128
L = 128
C = 40
NP = 10240

NC = 2
NS = 16
NW = NC * NS
FPT = L // NW
K = 128
CPT = 2560
EPAD = CPT * K
SEGC = 16
NSEG = CPT // SEGC


def _hop_body(xt_hbm, w_hbm, recv_hbm, send_hbm, out_hbm,
              ridx_v, sidx_v, w_all, xt_v, acc_v, isem):
    cid = lax.axis_index("c")
    sid = lax.axis_index("s")
    g = cid * NS + sid
    f0 = g * FPT

    pltpu.sync_copy(xt_hbm.at[pl.ds(f0, FPT)], xt_v)

    zvec = jnp.zeros((16,), jnp.float32)

    @pl.loop(0, FPT * NP // 16)
    def _zero(i):
        acc_v[0, pl.ds(pl.multiple_of(i * 16, 16), 16)] = zvec

    fvecs = [jnp.full((16,), f, jnp.int32) for f in range(FPT)]

    @pl.loop(0, NSEG)
    def _segment(s):
        sbase = s * SEGC
        c_r = pltpu.async_copy(recv_hbm.at[pl.ds(sbase, SEGC)], ridx_v, isem)
        c_s = pltpu.async_copy(send_hbm.at[pl.ds(sbase, SEGC)], sidx_v, isem)
        c_w = pltpu.async_copy(w_hbm.at[pl.ds(sbase, SEGC)], w_all, isem)
        c_r.wait()
        c_s.wait()
        c_w.wait()

        @pl.loop(0, SEGC)
        def _chunk(ic):
            for go in range(K // 16):
                sl = pl.ds(go * 16, 16)
                r16 = ridx_v[ic, sl]
                s16 = sidx_v[ic, sl]
                w16 = w_all[ic, sl]
                for f in range(FPT):
                    xv = plsc.load_gather(xt_v, [fvecs[f], r16])
                    plsc.addupdate_scatter(acc_v, [fvecs[f], s16], xv * w16)

    pltpu.sync_copy(acc_v, out_hbm.at[pl.ds(f0, FPT)])


_hop = functools.partial(
    pl.kernel,
    out_type=jax.ShapeDtypeStruct((L, NP), jnp.float32),
    mesh=plsc.VectorSubcoreMesh(core_axis_name="c", subcore_axis_name="s",
                                num_cores=NC, num_subcores=NS),
    compiler_params=pltpu.CompilerParams(use_tc_tiling_on_sc=False,
                                         needs_layout_passes=False),
    scratch_types=[
        pltpu.VMEM((SEGC, K), jnp.int32),
        pltpu.VMEM((SEGC, K), jnp.int32),
        pltpu.VMEM((SEGC, K), jnp.float32),
        pltpu.VMEM((FPT, NP), jnp.float32),
        pltpu.VMEM((FPT, NP), jnp.float32),
        pltpu.SemaphoreType.DMA,
    ],
)(_hop_body)


BM = 2048


def _enc_body(x_ref, w_ref, b_ref, o_ref):
    y = jnp.dot(w_ref[...], x_ref[...], preferred_element_type=jnp.float32)
    o_ref[...] = jnp.maximum(y + b_ref[...][:, :1], 0.0)


def _encoder_t(xt, wt, b):
    return pl.pallas_call(
        _enc_body,
        grid=(NP // BM,),
        in_specs=[
            pl.BlockSpec((D, BM), lambda i: (0, i)),
            pl.BlockSpec((L, D), lambda i: (0, 0)),
            pl.BlockSpec((L, 128), lambda i: (0, 0)),
        ],
        out_specs=pl.BlockSpec((L, BM), lambda i: (0, i)),
        out_shape=jax.ShapeDtypeStruct((L, NP), jnp.float32),
    )(xt, wt, b)


def _upd_body(p_ref, w_ref, b_ref, o_ref):
    conv = p_ref[...]
    h = jnp.dot(w_ref[...], conv, preferred_element_type=jnp.float32)
    o_ref[...] = jnp.maximum(h + b_ref[...][:, :1], 0.0) + conv


def _update_t(conv_t, wt, b):
    return pl.pallas_call(
        _upd_body,
        grid=(NP // BM,),
        in_specs=[
            pl.BlockSpec((L, BM), lambda i: (0, i)),
            pl.BlockSpec((L, L), lambda i: (0, 0)),
            pl.BlockSpec((L, 128), lambda i: (0, 0)),
        ],
        out_specs=pl.BlockSpec((L, BM), lambda i: (0, i)),
        out_shape=jax.ShapeDtypeStruct((L, NP), jnp.float32),
    )(conv_t, wt, b)


def _upd_dec_body(p_ref, w_ref, b_ref, dw_ref, db_ref, o_ref):
    conv = p_ref[...]
    h = jnp.dot(w_ref[...], conv, preferred_element_type=jnp.float32)
    x = jnp.maximum(h + b_ref[...][:, :1], 0.0) + conv
    o_ref[...] = jnp.dot(dw_ref[...], x,
                         preferred_element_type=jnp.float32) + db_ref[...][:, :1]


def _update_dec_t(conv_t, wt, b, dwt, db):
    return pl.pallas_call(
        _upd_dec_body,
        grid=(NP // BM,),
        in_specs=[
            pl.BlockSpec((L, BM), lambda i: (0, i)),
            pl.BlockSpec((L, L), lambda i: (0, 0)),
            pl.BlockSpec((L, 128), lambda i: (0, 0)),
            pl.BlockSpec((L, L), lambda i: (0, 0)),
            pl.BlockSpec((L, 128), lambda i: (0, 0)),
        ],
        out_specs=pl.BlockSpec((L, BM), lambda i: (0, i)),
        out_shape=jax.ShapeDtypeStruct((L, NP), jnp.float32),
    )(conv_t, wt, b, dwt, db)


def kernel(nodes, edges, senders, receivers, enc_W, enc_b, core0_W, core0_b,
           core1_W, core1_b, dec_W, dec_b):
    w = edges.reshape(E)
    senders = senders.astype(jnp.int32)
    receivers = receivers.astype(jnp.int32)

    npad = EPAD - E
    w_p = jnp.concatenate([w, jnp.zeros((npad,), jnp.float32)]
                          ).reshape(CPT, K)
    recv_p = jnp.concatenate([receivers, jnp.zeros((npad,), jnp.int32)]
                             ).reshape(CPT, K)
    send_p = jnp.concatenate([senders, jnp.zeros((npad,), jnp.int32)]
                             ).reshape(CPT, K)

    def col(b):
        return jnp.tile(b.reshape(-1, 1), (1, 128))

    nodes_tp = jnp.pad(nodes.T, ((0, 0), (0, NP - N)))
    xt = _encoder_t(nodes_tp, enc_W.T, col(enc_b))
    conv_t = _hop(xt, w_p, recv_p, send_p)
    xt = _update_t(conv_t, core0_W.T, col(core0_b))
    conv_t = _hop(xt, w_p, recv_p, send_p)

    dwt_pad = jnp.zeros((L, L), jnp.float32).at[:C, :].set(dec_W.T)
    db_pad = jnp.zeros((L,), jnp.float32).at[:C].set(dec_b)
    out_t = _update_dec_t(conv_t, core1_W.T, col(core1_b),
                          dwt_pad, col(db_pad))
    return out_t[:C, :N].T


# speedup vs baseline: 1.8759x; 1.8759x over previous
import functools

import jax
import jax.numpy as jnp
from jax import lax
from jax.experimental import pallas as pl
from jax.experimental.pallas import tpu as pltpu
from jax.experimental.pallas import tpu_sc as plsc

N = 10000
E = 320000
D = 128
L = 128
C = 40

NC = 2
NS = 16
NW = NC * NS
EPW = E // NW
K = 80
NCHUNK = EPW // K
NSIO = 10
RPS = N // NSIO
ZROWS = 200


def _hop_body(x_hbm, w_hbm, recv_hbm, send_hbm, out_hbm,
              ridx_v, sidx_v, w_v, rows_v, zb_v, acc_sh, sem):
    cid = lax.axis_index("c")
    sid = lax.axis_index("s")
    wid = cid * NS + sid

    zvec = jnp.zeros((16,), jnp.float32)

    @pl.when(sid < NSIO)
    def _zero():
        @pl.loop(0, ZROWS * (D // 16))
        def _zero_fill(i):
            r = i // (D // 16)
            c = i % (D // 16)
            zb_v[r, pl.ds(pl.multiple_of(c * 16, 16), 16)] = zvec

        @pl.loop(0, RPS // ZROWS)
        def _zero_acc(j):
            pltpu.sync_copy(zb_v,
                            acc_sh.at[pl.ds(sid * RPS + j * ZROWS, ZROWS)])

    plsc.subcore_barrier()

    base0 = wid * EPW

    @pl.loop(0, NCHUNK)
    def _chunk(i):
        base = base0 + i * K
        pltpu.sync_copy(recv_hbm.at[pl.ds(base, K)], ridx_v)
        pltpu.sync_copy(send_hbm.at[pl.ds(base, K)], sidx_v)
        pltpu.sync_copy(w_hbm.at[pl.ds(base, K)], w_v)
        pltpu.async_copy(x_hbm.at[ridx_v], rows_v, sem).wait()

        @pl.loop(0, K // 16)
        def _scale(g):
            w16 = w_v[pl.ds(pl.multiple_of(g * 16, 16), 16)]
            for t in range(16):
                wk = w16[t]
                for j in range(D // 16):
                    sl = pl.ds(j * 16, 16)
                    rows_v[g * 16 + t, sl] = rows_v[g * 16 + t, sl] * wk

        pltpu.sync_copy(rows_v, acc_sh.at[sidx_v], add=True)

    plsc.subcore_barrier()

    @pl.when(sid < NSIO)
    def _copy_out():
        pltpu.sync_copy(acc_sh.at[pl.ds(sid * RPS, RPS)],
                        out_hbm.at[cid, pl.ds(sid * RPS, RPS)])


_hop = functools.partial(
    pl.kernel,
    out_type=jax.ShapeDtypeStruct((NC, N, L), jnp.float32),
    mesh=plsc.VectorSubcoreMesh(core_axis_name="c", subcore_axis_name="s",
                                num_cores=NC, num_subcores=NS),
    scratch_types=[
        pltpu.VMEM((K,), jnp.int32),
        pltpu.VMEM((K,), jnp.int32),
        pltpu.VMEM((K,), jnp.float32),
        pltpu.VMEM((K, L), jnp.float32),
        pltpu.VMEM((ZROWS, L), jnp.float32),
        pltpu.VMEM_SHARED((N, L), jnp.float32),
        pltpu.SemaphoreType.DMA,
    ],
)(_hop_body)


BM = 2000


def _encoder_body(x_ref, w_ref, b_ref, o_ref):
    y = jnp.dot(x_ref[...], w_ref[...], preferred_element_type=jnp.float32)
    o_ref[...] = jnp.maximum(y + b_ref[...], 0.0)


def _encoder(x, w, b):
    return pl.pallas_call(
        _encoder_body,
        grid=(N // BM,),
        in_specs=[
            pl.BlockSpec((BM, D), lambda i: (i, 0)),
            pl.BlockSpec((D, L), lambda i: (0, 0)),
            pl.BlockSpec((1, L), lambda i: (0, 0)),
        ],
        out_specs=pl.BlockSpec((BM, L), lambda i: (i, 0)),
        out_shape=jax.ShapeDtypeStruct((N, L), jnp.float32),
    )(x, w, b.reshape(1, L))


def _update_body(p_ref, w_ref, b_ref, o_ref):
    conv = p_ref[0] + p_ref[1]
    h = jnp.dot(conv, w_ref[...], preferred_element_type=jnp.float32)
    o_ref[...] = jnp.maximum(h + b_ref[...], 0.0) + conv


def _update(parts, w, b):
    return pl.pallas_call(
        _update_body,
        grid=(N // BM,),
        in_specs=[
            pl.BlockSpec((NC, BM, L), lambda i: (0, i, 0)),
            pl.BlockSpec((L, L), lambda i: (0, 0)),
            pl.BlockSpec((1, L), lambda i: (0, 0)),
        ],
        out_specs=pl.BlockSpec((BM, L), lambda i: (i, 0)),
        out_shape=jax.ShapeDtypeStruct((N, L), jnp.float32),
    )(parts, w, b.reshape(1, L))


def _update_dec_body(p_ref, w_ref, b_ref, dw_ref, db_ref, o_ref):
    conv = p_ref[0] + p_ref[1]
    h = jnp.dot(conv, w_ref[...], preferred_element_type=jnp.float32)
    x = jnp.maximum(h + b_ref[...], 0.0) + conv
    o_ref[...] = jnp.dot(x, dw_ref[...],
                         preferred_element_type=jnp.float32) + db_ref[...]


def _update_dec(parts, w, b, dw, db):
    return pl.pallas_call(
        _update_dec_body,
        grid=(N // BM,),
        in_specs=[
            pl.BlockSpec((NC, BM, L), lambda i: (0, i, 0)),
            pl.BlockSpec((L, L), lambda i: (0, 0)),
            pl.BlockSpec((1, L), lambda i: (0, 0)),
            pl.BlockSpec((L, L), lambda i: (0, 0)),
            pl.BlockSpec((1, L), lambda i: (0, 0)),
        ],
        out_specs=pl.BlockSpec((BM, L), lambda i: (i, 0)),
        out_shape=jax.ShapeDtypeStruct((N, L), jnp.float32),
    )(parts, w, b.reshape(1, L), dw, db.reshape(1, L))


def kernel(nodes, edges, senders, receivers, enc_W, enc_b, core0_W, core0_b,
           core1_W, core1_b, dec_W, dec_b):
    w = edges.reshape(E)
    senders = senders.astype(jnp.int32)
    receivers = receivers.astype(jnp.int32)

    x = _encoder(nodes, enc_W, enc_b)
    parts = _hop(x, w, receivers, senders)
    x = _update(parts, core0_W, core0_b)
    parts = _hop(x, w, receivers, senders)

    dw_pad = jnp.zeros((L, L), jnp.float32).at[:, :C].set(dec_W)
    db_pad = jnp.zeros((L,), jnp.float32).at[:C].set(dec_b)
    out = _update_dec(parts, core1_W, core1_b, dw_pad, db_pad)
    return out[:, :C]
